# baseline (speedup 1.0000x reference)
import functools

import jax
import jax.numpy as jnp
from jax import lax
from jax.experimental import pallas as pl
from jax.experimental.pallas import tpu as pltpu
from jax.experimental.pallas import tpu_sc as plsc

B = 4096
L = 200
FEAT = 64
VOCAB = 1000000

NC = 2
NS = 16
NW = NC * NS
BPW = B // NW
LANES = 16
NV = FEAT // LANES
LC0 = 104
LC1 = L - LC0


def _sc_pool_body(inp_hbm, tbl_hbm, out_hbm, idx_v, rows_v, pooled_v, sem):
    wid = lax.axis_index("s") * NC + lax.axis_index("c")
    base = wid * BPW
    pltpu.sync_copy(inp_hbm.at[pl.ds(base, BPW)], idx_v)

    def row(b, carry):
        cp0 = pltpu.async_copy(
            tbl_hbm.at[idx_v.at[b, pl.ds(0, LC0)]],
            rows_v.at[pl.ds(0, LC0)], sem)
        cp1 = pltpu.async_copy(
            tbl_hbm.at[idx_v.at[b, pl.ds(LC0, LC1)]],
            rows_v.at[pl.ds(LC0, LC1)], sem)
        cp0.wait()
        cp1.wait()

        def red(i, acc):
            return tuple(acc[j] + rows_v[i, pl.ds(LANES * j, LANES)]
                         for j in range(NV))

        acc = lax.fori_loop(
            0, L, red, tuple(jnp.zeros((LANES,), jnp.float32)
                             for _ in range(NV)))
        inv_l = jnp.float32(1.0 / L)
        for j in range(NV):
            pooled_v[b, pl.ds(LANES * j, LANES)] = acc[j] * inv_l
        return carry

    lax.fori_loop(0, BPW, row, 0)
    pltpu.sync_copy(pooled_v, out_hbm.at[pl.ds(base, BPW)])


@jax.jit
def _sc_pool(idx, table):
    mesh = plsc.VectorSubcoreMesh(core_axis_name="c", subcore_axis_name="s")
    return pl.kernel(
        _sc_pool_body,
        out_type=jax.ShapeDtypeStruct((B, FEAT), jnp.float32),
        mesh=mesh,
        scratch_types=[
            pltpu.VMEM((BPW, L), jnp.int32),
            pltpu.VMEM((L, FEAT), jnp.float32),
            pltpu.VMEM((BPW, FEAT), jnp.float32),
            pltpu.SemaphoreType.DMA,
        ],
        compiler_params=pltpu.CompilerParams(use_tc_tiling_on_sc=False),
    )(idx, table)


def _tc_head_body(x_ref, w_ref, b_ref, bng_ref, bnb_ref, lng_ref, lnb_ref,
                  out_ref):
    x = x_ref[...]
    w = w_ref[...]
    y = lax.dot_general(x, w, (((1,), (1,)), ((), ())),
                        preferred_element_type=jnp.float32,
                        precision=lax.Precision.HIGHEST)
    y = y + b_ref[...]
    mean = jnp.mean(y, axis=0, keepdims=True)
    var = jnp.mean(jnp.square(y - mean), axis=0, keepdims=True)
    y = (y - mean) * lax.rsqrt(var + 1e-5) * bng_ref[...] + bnb_ref[...]
    mu = jnp.mean(y, axis=1, keepdims=True)
    v = jnp.mean(jnp.square(y - mu), axis=1, keepdims=True)
    out_ref[...] = (y - mu) * lax.rsqrt(v + 1e-5) * lng_ref[...] + lnb_ref[...]


@jax.jit
def _tc_head(pooled, fc_w, fc_b, bn_g, bn_b, ln_g, ln_b):
    r = lambda p: p.reshape(1, FEAT)
    return pl.pallas_call(
        _tc_head_body,
        out_shape=jax.ShapeDtypeStruct((B, FEAT), jnp.float32),
    )(pooled, fc_w, r(fc_b), r(bn_g), r(bn_b), r(ln_g), r(ln_b))


def kernel(input_data, emb_table, fc_w, fc_b, bn_g, bn_b, ln_g, ln_b):
    idx = input_data.astype(jnp.int32)
    pooled = _sc_pool(idx, emb_table)
    return _tc_head(pooled, fc_w, fc_b, bn_g, bn_b, ln_g, ln_b)


# speedup vs baseline: 1.1746x; 1.1746x over previous
import functools

import jax
import jax.numpy as jnp
from jax import lax
from jax.experimental import pallas as pl
from jax.experimental.pallas import tpu as pltpu
from jax.experimental.pallas import tpu_sc as plsc

B = 4096
L = 200
FEAT = 64
VOCAB = 1000000

NC = 2
NS = 16
NW = NC * NS
BPW = B // NW
LANES = 16
NV = FEAT // LANES
LC0 = 104
LC1 = L - LC0


NBUF = 2
UNROLL = 8


def _sc_pool_body(inp_hbm, tbl_hbm, out_hbm, idx_v, rows_v, pooled_v, sems):
    wid = lax.axis_index("s") * NC + lax.axis_index("c")
    base = wid * BPW
    pltpu.sync_copy(inp_hbm.at[pl.ds(base, BPW)], idx_v)

    def fire(r, s):
        @pl.when(r < BPW)
        def _():
            pltpu.async_copy(
                tbl_hbm.at[idx_v.at[r, pl.ds(0, LC0)]],
                rows_v.at[s, pl.ds(0, LC0)], sems.at[s])
            pltpu.async_copy(
                tbl_hbm.at[idx_v.at[r, pl.ds(LC0, LC1)]],
                rows_v.at[s, pl.ds(LC0, LC1)], sems.at[s])

    def drain(s):
        pltpu.make_async_copy(
            tbl_hbm.at[pl.ds(0, L)], rows_v.at[s], sems.at[s]).wait()

    def reduce_store(r, s):
        nacc = 2 * NV

        def red(i, acc):
            acc = list(acc)
            for u in range(UNROLL):
                row = i * UNROLL + u
                for j in range(NV):
                    k = (u % 2) * NV + j
                    acc[k] = acc[k] + rows_v[s, row, pl.ds(LANES * j, LANES)]
            return tuple(acc)

        acc = lax.fori_loop(
            0, L // UNROLL, red,
            tuple(jnp.zeros((LANES,), jnp.float32) for _ in range(nacc)))
        inv_l = jnp.float32(1.0 / L)
        for j in range(NV):
            pooled_v[r, pl.ds(LANES * j, LANES)] = (acc[j] + acc[NV + j]) * inv_l

    for s in range(NBUF):
        fire(jnp.int32(s), s)

    def body(g, carry):
        for s in range(NBUF):
            r = g * NBUF + s
            drain(s)
            reduce_store(r, s)
            fire(r + NBUF, s)
        return carry

    lax.fori_loop(0, BPW // NBUF, body, 0)
    pltpu.sync_copy(pooled_v, out_hbm.at[pl.ds(base, BPW)])


@jax.jit
def _sc_pool(idx, table):
    mesh = plsc.VectorSubcoreMesh(core_axis_name="c", subcore_axis_name="s")
    return pl.kernel(
        _sc_pool_body,
        out_type=jax.ShapeDtypeStruct((B, FEAT), jnp.float32),
        mesh=mesh,
        scratch_types=[
            pltpu.VMEM((BPW, L), jnp.int32),
            pltpu.VMEM((NBUF, L, FEAT), jnp.float32),
            pltpu.VMEM((BPW, FEAT), jnp.float32),
            pltpu.SemaphoreType.DMA((NBUF,)),
        ],
        compiler_params=pltpu.CompilerParams(use_tc_tiling_on_sc=False),
    )(idx, table)


def _tc_head_body(x_ref, w_ref, b_ref, bng_ref, bnb_ref, lng_ref, lnb_ref,
                  out_ref):
    x = x_ref[...]
    w = w_ref[...]
    y = lax.dot_general(x, w, (((1,), (1,)), ((), ())),
                        preferred_element_type=jnp.float32,
                        precision=lax.Precision.HIGHEST)
    y = y + b_ref[...]
    mean = jnp.mean(y, axis=0, keepdims=True)
    var = jnp.mean(jnp.square(y - mean), axis=0, keepdims=True)
    y = (y - mean) * lax.rsqrt(var + 1e-5) * bng_ref[...] + bnb_ref[...]
    mu = jnp.mean(y, axis=1, keepdims=True)
    v = jnp.mean(jnp.square(y - mu), axis=1, keepdims=True)
    out_ref[...] = (y - mu) * lax.rsqrt(v + 1e-5) * lng_ref[...] + lnb_ref[...]


@jax.jit
def _tc_head(pooled, fc_w, fc_b, bn_g, bn_b, ln_g, ln_b):
    r = lambda p: p.reshape(1, FEAT)
    return pl.pallas_call(
        _tc_head_body,
        out_shape=jax.ShapeDtypeStruct((B, FEAT), jnp.float32),
    )(pooled, fc_w, r(fc_b), r(bn_g), r(bn_b), r(ln_g), r(ln_b))


def kernel(input_data, emb_table, fc_w, fc_b, bn_g, bn_b, ln_g, ln_b):
    idx = input_data.astype(jnp.int32)
    pooled = _sc_pool(idx, emb_table)
    return _tc_head(pooled, fc_w, fc_b, bn_g, bn_b, ln_g, ln_b)
